# Initial kernel scaffold; baseline (speedup 1.0000x reference)
import jax
import jax.numpy as jnp
from jax.experimental import pallas as pl


def kernel(boxes, scores):
    raise NotImplementedError("write your pallas kernel here")


# speedup vs baseline: 30.6056x; 30.6056x over previous
import functools

import jax
import jax.numpy as jnp
from jax.experimental import pallas as pl
from jax.experimental.pallas import tpu as pltpu

IMAGE_SIZE = 1280
CONF_THRESH = 0.55
IOU_THRESH = 0.5
MAX_OUT = 100
N_BOXES = 20000

_ROWS = 160
_NPAD = _ROWS * 128


def _nms_body(x1_ref, y1_ref, x2_ref, y2_ref, s_ref, out_ref):
    s0 = s_ref[:]
    sm0 = jnp.where(s0 > CONF_THRESH, s0, -1.0)
    iota_i = (jax.lax.broadcasted_iota(jnp.int32, (_ROWS, 128), 0) * 128
              + jax.lax.broadcasted_iota(jnp.int32, (_ROWS, 128), 1))
    lane = jax.lax.broadcasted_iota(jnp.int32, (1, 128), 1)
    x1 = x1_ref[:]
    y1 = y1_ref[:]
    x2 = x2_ref[:]
    y2 = y2_ref[:]

    def step(i, sm):
        m = jnp.max(sm)
        idx = jnp.min(jnp.where(sm >= m, iota_i, jnp.int32(2**30)))
        sel = iota_i == idx
        bx1 = jnp.max(jnp.where(sel, x1, -3.0e7))
        by1 = jnp.max(jnp.where(sel, y1, -3.0e7))
        bx2 = jnp.max(jnp.where(sel, x2, -3.0e7))
        by2 = jnp.max(jnp.where(sel, y2, -3.0e7))
        ok = m > 0.0
        row = (jnp.where(lane == 0, bx1, 0.0) + jnp.where(lane == 1, by1, 0.0)
               + jnp.where(lane == 2, bx2, 0.0) + jnp.where(lane == 3, by2, 0.0)
               + jnp.where(lane == 4, m, 0.0))
        out_ref[pl.ds(i, 1), :] = jnp.where(ok, row, 0.0)
        ix1 = jnp.maximum(bx1, x1)
        iy1 = jnp.maximum(by1, y1)
        ix2 = jnp.minimum(bx2, x2)
        iy2 = jnp.minimum(by2, y2)
        inter = jnp.maximum(ix2 - ix1, 0.0) * jnp.maximum(iy2 - iy1, 0.0)
        area_a = (bx2 - bx1) * (by2 - by1)
        area_b = (x2 - x1) * (y2 - y1)
        iou = inter / (area_a + area_b - inter + 1e-9)
        return jnp.where(iou > IOU_THRESH, -1.0, sm)

    jax.lax.fori_loop(0, MAX_OUT, step, sm0)


@jax.jit
def kernel(boxes, scores):
    padn = _NPAD - N_BOXES
    bt = jnp.pad(boxes, ((0, padn), (0, 0))).T.reshape(4, _ROWS, 128)
    sp = jnp.pad(scores, (0, padn), constant_values=-1.0).reshape(_ROWS, 128)
    out = pl.pallas_call(
        _nms_body,
        out_shape=jax.ShapeDtypeStruct((MAX_OUT, 128), jnp.float32),
    )(bt[0], bt[1], bt[2], bt[3], sp)
    return out[:, :5]
